# baseline (speedup 1.0000x reference)
import functools

import jax
import jax.numpy as jnp
from jax import lax
from jax.experimental import pallas as pl
from jax.experimental.pallas import tpu as pltpu
from jax.experimental.pallas import tpu_sc as plsc

N_PRIM = 64
VIEW = 4096
HIDDEN = 4096
TOPK = 8
TOKENS = 8192

M_TILE = 512
H_TILE = 2048

NC, NS, LANES = 2, 16, 16
NW = NC * NS
TOKS_PER_W = TOKENS // NW


def _mlp_kernel(z_ref, w1_hbm, b1_ref, w2_ref, b2_ref, out_ref,
                acc_ref, w1_vmem, dma_sem):
    h_idx = pl.program_id(0)
    n_h = pl.num_programs(0)
    m_idx = pl.program_id(1)
    n_m = pl.num_programs(1) - 1

    @pl.when(m_idx == 0)
    def _fetch_w1():
        pltpu.make_async_copy(
            w1_hbm.at[pl.ds(h_idx * H_TILE, H_TILE), :], w1_vmem, dma_sem
        ).start()
        pltpu.make_async_copy(
            w1_hbm.at[pl.ds(h_idx * H_TILE, H_TILE), :], w1_vmem, dma_sem
        ).wait()

    @pl.when(m_idx < n_m)
    def _compute():
        rows = pl.ds(m_idx * M_TILE, M_TILE)
        h = jnp.dot(z_ref[...], w1_vmem[...].T,
                    preferred_element_type=jnp.float32)
        h = h + b1_ref[...]
        h = 0.5 * h * (1.0 + jax.lax.erf(h * 0.7071067811865476))
        partial = jnp.dot(h, w2_ref[...].T, preferred_element_type=jnp.float32)

        @pl.when(h_idx == 0)
        def _init():
            acc_ref[rows, :] = partial + b2_ref[...]

        @pl.when(h_idx != 0)
        def _accum():
            acc_ref[rows, :] = acc_ref[rows, :] + partial

    @pl.when(jnp.logical_and(h_idx == n_h - 1, m_idx > 0))
    def _logits_out():
        prev_rows = pl.ds((m_idx - 1) * M_TILE, M_TILE)
        out_ref[...] = acc_ref[prev_rows, :].T


def _mlp_probs_t(z, W1, b1_2d, W2, b2_2d):
    tokens = z.shape[0]
    n_h = HIDDEN // H_TILE
    n_m = tokens // M_TILE
    last_m = n_m - 1
    grid = (n_h, n_m + 1)
    return pl.pallas_call(
        _mlp_kernel,
        grid=grid,
        in_specs=[
            pl.BlockSpec((M_TILE, VIEW), lambda h, m: (jnp.minimum(m, last_m), 0)),
            pl.BlockSpec(memory_space=pl.ANY),
            pl.BlockSpec((1, H_TILE), lambda h, m: (0, h)),
            pl.BlockSpec((N_PRIM, H_TILE), lambda h, m: (0, h)),
            pl.BlockSpec((1, N_PRIM), lambda h, m: (0, 0)),
        ],
        out_specs=pl.BlockSpec(
            (N_PRIM, M_TILE),
            lambda h, m: (0, jnp.maximum(m, 1) - 1),
        ),
        out_shape=jax.ShapeDtypeStruct((N_PRIM, tokens), jnp.float32),
        scratch_shapes=[
            pltpu.VMEM((tokens, N_PRIM), jnp.float32),
            pltpu.VMEM((H_TILE, VIEW), jnp.float32),
            pltpu.SemaphoreType.DMA,
        ],
    )(z, W1, b1_2d, W2, b2_2d)


def _sc_route_t(probs_t):
    mesh = plsc.VectorSubcoreMesh(core_axis_name="c", subcore_axis_name="s")

    @functools.partial(
        pl.kernel,
        mesh=mesh,
        out_type=jax.ShapeDtypeStruct((N_PRIM, TOKENS), jnp.float32),
        scratch_types=[
            pltpu.VMEM((N_PRIM, TOKS_PER_W), jnp.float32),
            pltpu.VMEM((N_PRIM, TOKS_PER_W), jnp.float32),
            pltpu.SemaphoreType.DMA,
        ],
    )
    def k(p_hbm, o_hbm, pbuf, obuf, sem):
        wid = lax.axis_index("s") * NC + lax.axis_index("c")
        base = wid * TOKS_PER_W
        pltpu.async_copy(
            p_hbm.at[:, pl.ds(base, TOKS_PER_W)], pbuf, sem
        ).wait()

        @pl.loop(0, TOKS_PER_W, step=LANES)
        def _(t0):
            sl = pl.ds(t0, LANES)
            logit = [pbuf[p, sl] for p in range(N_PRIM)]
            m0 = logit[0]
            for p in range(1, N_PRIM):
                m0 = jnp.maximum(m0, logit[p])
            e = [jnp.exp(l - m0) for l in logit]
            esum = e[0]
            for p in range(1, N_PRIM):
                esum = esum + e[p]
            cur = list(logit)
            for _ in range(TOPK):
                m = cur[0]
                for p in range(1, N_PRIM):
                    m = jnp.maximum(m, cur[p])
                taken = m * 0.0
                for p in range(N_PRIM):
                    hit = jnp.where(cur[p] == m, 1.0 - taken, 0.0)
                    cur[p] = jnp.where(hit > 0.5, -jnp.inf, cur[p])
                    taken = jnp.maximum(taken, hit)
            sel = [jnp.where(c == -jnp.inf, ev, 0.0)
                   for c, ev in zip(cur, e)]
            total = sel[0]
            for p in range(1, N_PRIM):
                total = total + sel[p]
            total = total + 1e-8 * esum
            for p in range(N_PRIM):
                obuf[p, sl] = sel[p] / total

        pltpu.sync_copy(obuf, o_hbm.at[:, pl.ds(base, TOKS_PER_W)])

    return k(probs_t)


@functools.partial(jax.jit, static_argnames=())
def kernel(z, W1, b1, W2, b2):
    b1_2d = b1.reshape(1, HIDDEN)
    b2_2d = b2.reshape(1, N_PRIM)
    probs_t = _mlp_probs_t(z, W1, b1_2d, W2, b2_2d)
    return _sc_route_t(probs_t).T


# speedup vs baseline: 1.1397x; 1.1397x over previous
import functools

import jax
import jax.numpy as jnp
from jax.experimental import pallas as pl
from jax.experimental.pallas import tpu as pltpu

N_PRIM = 64
VIEW = 4096
HIDDEN = 4096
TOPK = 8
TOKENS = 8192

M_TILE = 512
H_TILE = 2048


def _router_kernel(z_ref, w1_hbm, b1_ref, w2_ref, b2_ref, out_ref,
                   acc_ref, w1_vmem, dma_sem):
    h_idx = pl.program_id(0)
    n_h = pl.num_programs(0)
    m_idx = pl.program_id(1)
    n_m = pl.num_programs(1) - 1

    @pl.when(m_idx == 0)
    def _fetch_w1():
        pltpu.make_async_copy(
            w1_hbm.at[pl.ds(h_idx * H_TILE, H_TILE), :], w1_vmem, dma_sem
        ).start()
        pltpu.make_async_copy(
            w1_hbm.at[pl.ds(h_idx * H_TILE, H_TILE), :], w1_vmem, dma_sem
        ).wait()

    @pl.when(m_idx < n_m)
    def _compute():
        rows = pl.ds(m_idx * M_TILE, M_TILE)
        h = jnp.dot(z_ref[...], w1_vmem[...].T,
                    preferred_element_type=jnp.float32)
        h = h + b1_ref[...]
        h = 0.5 * h * (1.0 + jax.lax.erf(h * 0.7071067811865476))
        partial = jnp.dot(h, w2_ref[...].T, preferred_element_type=jnp.float32)

        @pl.when(h_idx == 0)
        def _init():
            acc_ref[rows, :] = partial + b2_ref[...]

        @pl.when(h_idx != 0)
        def _accum():
            acc_ref[rows, :] = acc_ref[rows, :] + partial

    @pl.when(jnp.logical_and(h_idx == n_h - 1, m_idx > 0))
    def _finalize():
        prev_rows = pl.ds((m_idx - 1) * M_TILE, M_TILE)
        logits = acc_ref[prev_rows, :]
        mx0 = jnp.max(logits, axis=-1, keepdims=True)
        e = jnp.exp(logits - mx0)
        probs = e / jnp.sum(e, axis=-1, keepdims=True)
        lane = jax.lax.broadcasted_iota(
            jnp.int32, logits.shape, 1).astype(jnp.float32)
        cur = logits
        for _ in range(TOPK):
            mx = jnp.max(cur, axis=-1, keepdims=True)
            first_lane = jnp.min(
                jnp.where(cur == mx, lane, float(N_PRIM)),
                axis=-1, keepdims=True,
            )
            cur = jnp.where(lane == first_lane, -jnp.inf, cur)
        sparse = jnp.where(cur == -jnp.inf, probs, 0.0)
        denom = jnp.sum(sparse, axis=-1, keepdims=True) + 1e-8
        out_ref[...] = sparse / denom


def _router_call(z, W1, b1_2d, W2, b2_2d):
    tokens = z.shape[0]
    n_h = HIDDEN // H_TILE
    n_m = tokens // M_TILE
    last_m = n_m - 1
    grid = (n_h, n_m + 1)
    return pl.pallas_call(
        _router_kernel,
        grid=grid,
        in_specs=[
            pl.BlockSpec((M_TILE, VIEW), lambda h, m: (jnp.minimum(m, last_m), 0)),
            pl.BlockSpec(memory_space=pl.ANY),
            pl.BlockSpec((1, H_TILE), lambda h, m: (0, h)),
            pl.BlockSpec((N_PRIM, H_TILE), lambda h, m: (0, h)),
            pl.BlockSpec((1, N_PRIM), lambda h, m: (0, 0)),
        ],
        out_specs=pl.BlockSpec(
            (M_TILE, N_PRIM),
            lambda h, m: (jnp.maximum(m, 1) - 1, 0),
        ),
        out_shape=jax.ShapeDtypeStruct((tokens, N_PRIM), jnp.float32),
        scratch_shapes=[
            pltpu.VMEM((tokens, N_PRIM), jnp.float32),
            pltpu.VMEM((H_TILE, VIEW), jnp.float32),
            pltpu.SemaphoreType.DMA,
        ],
    )(z, W1, b1_2d, W2, b2_2d)


@functools.partial(jax.jit, static_argnames=())
def kernel(z, W1, b1, W2, b2):
    b1_2d = b1.reshape(1, HIDDEN)
    b2_2d = b2.reshape(1, N_PRIM)
    return _router_call(z, W1, b1_2d, W2, b2_2d)
